# baseline (speedup 1.0000x reference)
import functools

import jax
import jax.numpy as jnp
from jax import lax
from jax.experimental import pallas as pl
from jax.experimental.pallas import tpu as pltpu
from jax.experimental.pallas import tpu_sc as plsc

_EPS = 1e-12
_K = 128
_NL = 16
_NBUF = 4


def _build_sc_gather(vocab, hid, tok, seq, nc, ns):
    nw = nc * ns
    per_w = tok // nw
    nj = per_w // _K
    posn = seq + _K
    assert per_w % _K == 0 and per_w % seq == 0 and nj >= _NBUF
    nfull = (nj // _NBUF) * _NBUF
    nk = hid // _NL

    mesh = plsc.VectorSubcoreMesh(core_axis_name="c", subcore_axis_name="s")

    @functools.partial(
        pl.kernel,
        mesh=mesh,
        out_type=jax.ShapeDtypeStruct((tok, hid), jnp.float32),
        scratch_types=[
            pltpu.VMEM((nj, _K), jnp.int32),
            pltpu.VMEM((per_w,), jnp.int32),
            pltpu.VMEM((posn * hid,), jnp.float32),
            pltpu.VMEM((hid,), jnp.float32),
            [pltpu.VMEM((_K, hid), jnp.float32) for _ in range(_NBUF)],
            [pltpu.SemaphoreType.DMA for _ in range(_NBUF)],
            [pltpu.SemaphoreType.DMA for _ in range(_NBUF)],
        ],
    )
    def sc_gather(table, idx, sid, pos_ext, dseg, out, idx_v, sid_v, pos_v,
                  dseg_v, bufs, gsems, osems):
        wid = lax.axis_index("s") * nc + lax.axis_index("c")
        base = wid * per_w
        pltpu.sync_copy(idx.at[wid], idx_v)
        pltpu.sync_copy(sid.at[wid], sid_v)
        pltpu.sync_copy(pos_ext, pos_v)
        pltpu.sync_copy(dseg, dseg_v)
        dsv = tuple(dseg_v[pl.ds(k * _NL, _NL)] for k in range(nk))

        def start_gather(j, b):
            pltpu.make_async_copy(table.at[idx_v.at[j]], bufs[b], gsems[b]).start()

        def wait_gather(b):
            pltpu.make_async_copy(table.at[idx_v.at[0]], bufs[b], gsems[b]).wait()

        def start_out(j, b):
            pltpu.make_async_copy(
                bufs[b], out.at[pl.ds(base + j * _K, _K)], osems[b]
            ).start()

        def wait_out(b):
            pltpu.make_async_copy(
                bufs[b], out.at[pl.ds(base, _K)], osems[b]
            ).wait()

        def chunk_add(j, b, dsv):
            buf = bufs[b]
            l0 = lax.rem(j * _K, seq)

            def grp(g, dsv):
                riv = sid_v[pl.ds(j * _K + g * _NL, _NL)].astype(jnp.float32)
                for u in range(_NL):
                    r = g * _NL + u
                    sf = riv[u]
                    poff = (l0 + r) * hid
                    for k in range(nk):
                        pv = pos_v[pl.ds(poff + k * _NL, _NL)]
                        plsc.addupdate(
                            buf.at[r, pl.ds(k * _NL, _NL)], pv + sf * dsv[k]
                        )
                return dsv

            return lax.fori_loop(0, _K // _NL, grp, dsv)

        def half_step(j, b, dsv):
            ab = (b + _NBUF - 1) % _NBUF
            gb = (b + _NBUF - 2) % _NBUF
            wait_gather(b)
            dsv = chunk_add(j, b, dsv)
            start_out(j, b)

            @pl.when(jnp.logical_and(j >= 1, j + 2 < nj))
            def _():
                wait_out(ab)

            @pl.when(j + 2 < nj)
            def _():
                start_gather(j + 2, gb)

            return dsv

        for b in range(2):
            start_gather(b, b)

        def step(i, dsv):
            for b in range(_NBUF):
                dsv = half_step(i * _NBUF + b, b, dsv)
            return dsv

        dsv = lax.fori_loop(0, nfull // _NBUF, step, dsv)
        for j in range(nfull, nj):
            dsv = half_step(j, j % _NBUF, dsv)
        for j in range(nj - 3, nj):
            wait_out(j % _NBUF)

    return sc_gather, nw, nj


def _ln_body(x_ref, w_ref, b_ref, o_ref):
    x = x_ref[...]
    mu = jnp.mean(x, axis=-1, keepdims=True)
    d = x - mu
    var = jnp.mean(d * d, axis=-1, keepdims=True)
    inv = lax.rsqrt(var + _EPS)
    o_ref[...] = d * inv * w_ref[0][None, :] + b_ref[0][None, :]


def kernel(token_ids, segment_ids, word_emb, pos_emb, seg_emb, ln_w, ln_b):
    bsz, seq = token_ids.shape
    vocab, hid = word_emb.shape
    nseg = seg_emb.shape[0]
    assert nseg == 2
    tok = bsz * seq

    info = plsc.get_sparse_core_info()
    nc, ns = info.num_cores, info.num_subcores
    sc_gather, nw, nj = _build_sc_gather(vocab, hid, tok, seq, nc, ns)

    idx = token_ids.astype(jnp.int32).reshape(nw, nj, _K)
    posseg0 = pos_emb[:seq] + seg_emb[0][None, :]
    pos_ext = jnp.concatenate([posseg0, posseg0[:_K]], axis=0).reshape(-1)
    dseg = seg_emb[1] - seg_emb[0]
    sid = segment_ids.astype(jnp.int32).reshape(nw, tok // nw)
    summed = sc_gather(word_emb, idx, sid, pos_ext, dseg)

    tb = 20480
    assert tok % tb == 0
    out = pl.pallas_call(
        _ln_body,
        grid=(tok // tb,),
        in_specs=[
            pl.BlockSpec((tb, hid), lambda i: (i, 0)),
            pl.BlockSpec((1, hid), lambda i: (0, 0)),
            pl.BlockSpec((1, hid), lambda i: (0, 0)),
        ],
        out_specs=pl.BlockSpec((tb, hid), lambda i: (i, 0)),
        out_shape=jax.ShapeDtypeStruct((tok, hid), jnp.float32),
        compiler_params=pltpu.CompilerParams(dimension_semantics=("parallel",)),
    )(summed, ln_w.reshape(1, hid), ln_b.reshape(1, hid))
    return out.reshape(bsz, seq, hid)


# speedup vs baseline: 1.2150x; 1.2150x over previous
import functools

import jax
import jax.numpy as jnp
from jax import lax
from jax.experimental import pallas as pl
from jax.experimental.pallas import tpu as pltpu
from jax.experimental.pallas import tpu_sc as plsc

_EPS = 1e-12
_K = 128
_NBUF = 4
_ZB = 25


def _build_sc_gather(vocab, hid, tok, nc, ns):
    nw = nc * ns
    per_w = tok // nw
    nj = per_w // _K
    assert per_w % _K == 0 and nj >= _NBUF
    nfull = (nj // _NBUF) * _NBUF

    mesh = plsc.VectorSubcoreMesh(core_axis_name="c", subcore_axis_name="s")

    @functools.partial(
        pl.kernel,
        mesh=mesh,
        out_type=jax.ShapeDtypeStruct((tok, hid), jnp.float32),
        scratch_types=[
            pltpu.VMEM((nj, _K), jnp.int32),
            [pltpu.VMEM((_K, hid), jnp.float32) for _ in range(_NBUF)],
            [pltpu.SemaphoreType.DMA for _ in range(_NBUF)],
            [pltpu.SemaphoreType.DMA for _ in range(_NBUF)],
        ],
    )
    def sc_gather(table, idx, out, idx_v, bufs, gsems, osems):
        wid = lax.axis_index("s") * nc + lax.axis_index("c")
        base = wid * per_w
        pltpu.sync_copy(idx.at[wid], idx_v)

        def start_gather(j, b):
            pltpu.make_async_copy(table.at[idx_v.at[j]], bufs[b], gsems[b]).start()

        def wait_gather(b):
            pltpu.make_async_copy(table.at[idx_v.at[0]], bufs[b], gsems[b]).wait()

        def start_out(j, b):
            pltpu.make_async_copy(
                bufs[b], out.at[pl.ds(base + j * _K, _K)], osems[b]
            ).start()

        def wait_out(b):
            pltpu.make_async_copy(
                bufs[b], out.at[pl.ds(base, _K)], osems[b]
            ).wait()

        def half_step(j, b):
            ab = (b + _NBUF - 1) % _NBUF
            gb = (b + _NBUF - 2) % _NBUF
            wait_gather(b)
            start_out(j, b)

            @pl.when(jnp.logical_and(j >= 1, j + 2 < nj))
            def _():
                wait_out(ab)

            @pl.when(j + 2 < nj)
            def _():
                start_gather(j + 2, gb)

        for b in range(2):
            start_gather(b, b)

        def step(i, carry):
            for b in range(_NBUF):
                half_step(i * _NBUF + b, b)
            return carry

        lax.fori_loop(0, nfull // _NBUF, step, 0)
        for j in range(nfull, nj):
            half_step(j, j % _NBUF)
        for j in range(nj - 3, nj):
            wait_out(j % _NBUF)

    return sc_gather, nw, nj


def _ln_body(x_ref, sid_ref, pos_ref, dseg_ref, w_ref, b_ref, o_ref):
    dseg = dseg_ref[...]
    w = w_ref[0][None, :]
    bb = b_ref[0][None, :]
    for z in range(_ZB):
        sg = sid_ref[z]
        seg = lax.dot_general(
            sg, dseg, (((0,), (0,)), ((), ())),
            preferred_element_type=jnp.float32,
        )
        e = x_ref[z] + pos_ref[z] + seg
        mu = jnp.mean(e, axis=-1, keepdims=True)
        d = e - mu
        var = jnp.mean(d * d, axis=-1, keepdims=True)
        inv = lax.rsqrt(var + _EPS)
        o_ref[z] = d * inv * w + bb


def kernel(token_ids, segment_ids, word_emb, pos_emb, seg_emb, ln_w, ln_b):
    bsz, seq = token_ids.shape
    vocab, hid = word_emb.shape
    nseg = seg_emb.shape[0]
    assert nseg == 2
    tok = bsz * seq

    info = plsc.get_sparse_core_info()
    nc, ns = info.num_cores, info.num_subcores
    sc_gather, nw, nj = _build_sc_gather(vocab, hid, tok, nc, ns)

    idx = token_ids.astype(jnp.int32).reshape(nw, nj, _K)
    gathered = sc_gather(word_emb, idx)

    g = tok // _K
    assert g % _ZB == 0 and (_ZB * _K) % seq == 0
    posseg0 = pos_emb[:seq] + seg_emb[0][None, :]
    pidx = (jnp.arange(_ZB * _K, dtype=jnp.int32) % seq).reshape(_ZB, _K)
    pos3 = jnp.take(posseg0, pidx, axis=0)
    sid3 = segment_ids.astype(jnp.float32).reshape(g, 1, _K)
    dseg = (seg_emb[1] - seg_emb[0]).reshape(1, hid)

    out = pl.pallas_call(
        _ln_body,
        grid=(g // _ZB,),
        in_specs=[
            pl.BlockSpec((_ZB, _K, hid), lambda i: (i, 0, 0)),
            pl.BlockSpec((_ZB, 1, _K), lambda i: (i, 0, 0)),
            pl.BlockSpec((_ZB, _K, hid), lambda i: (0, 0, 0)),
            pl.BlockSpec((1, hid), lambda i: (0, 0)),
            pl.BlockSpec((1, hid), lambda i: (0, 0)),
            pl.BlockSpec((1, hid), lambda i: (0, 0)),
        ],
        out_specs=pl.BlockSpec((_ZB, _K, hid), lambda i: (i, 0, 0)),
        out_shape=jax.ShapeDtypeStruct((g, _K, hid), jnp.float32),
        compiler_params=pltpu.CompilerParams(dimension_semantics=("parallel",)),
    )(
        gathered.reshape(g, _K, hid),
        sid3,
        pos3,
        dseg,
        ln_w.reshape(1, hid),
        ln_b.reshape(1, hid),
    )
    return out.reshape(bsz, seq, hid)


# speedup vs baseline: 1.6658x; 1.3710x over previous
import functools

import jax
import jax.numpy as jnp
from jax import lax
from jax.experimental import pallas as pl
from jax.experimental.pallas import tpu as pltpu
from jax.experimental.pallas import tpu_sc as plsc

_EPS = 1e-12
_K = 128
_NBUF = 4
_ZB = 25


def _build_sc_gather(vocab, hid, tok, nc, ns):
    nw = nc * ns
    per_w = tok // nw
    nj = per_w // _K
    assert per_w % _K == 0 and nj >= _NBUF
    nfull = (nj // _NBUF) * _NBUF

    mesh = plsc.VectorSubcoreMesh(core_axis_name="c", subcore_axis_name="s")

    @functools.partial(
        pl.kernel,
        mesh=mesh,
        out_type=jax.ShapeDtypeStruct((tok, hid), jnp.float32),
        scratch_types=[
            pltpu.VMEM((nj, _K), jnp.int32),
            [pltpu.VMEM((_K, hid), jnp.float32) for _ in range(_NBUF)],
            [pltpu.SemaphoreType.DMA for _ in range(_NBUF)],
            [pltpu.SemaphoreType.DMA for _ in range(_NBUF)],
        ],
    )
    def sc_gather(table, idx, out, idx_v, bufs, gsems, osems):
        wid = lax.axis_index("s") * nc + lax.axis_index("c")
        base = wid * per_w
        pltpu.sync_copy(idx.at[wid], idx_v)

        def start_gather(j, b):
            pltpu.make_async_copy(table.at[idx_v.at[j]], bufs[b], gsems[b]).start()

        def wait_gather(b):
            pltpu.make_async_copy(table.at[idx_v.at[0]], bufs[b], gsems[b]).wait()

        def start_out(j, b):
            pltpu.make_async_copy(
                bufs[b], out.at[pl.ds(base + j * _K, _K)], osems[b]
            ).start()

        def wait_out(b):
            pltpu.make_async_copy(
                bufs[b], out.at[pl.ds(base, _K)], osems[b]
            ).wait()

        def half_step(j, b):
            ab = (b + _NBUF - 1) % _NBUF
            gb = (b + _NBUF - 2) % _NBUF
            wait_gather(b)
            start_out(j, b)

            @pl.when(jnp.logical_and(j >= 1, j + 2 < nj))
            def _():
                wait_out(ab)

            @pl.when(j + 2 < nj)
            def _():
                start_gather(j + 2, gb)

        for b in range(2):
            start_gather(b, b)

        def step(i, carry):
            for b in range(_NBUF):
                half_step(i * _NBUF + b, b)
            return carry

        lax.fori_loop(0, nfull // _NBUF, step, 0)
        for j in range(nfull, nj):
            half_step(j, j % _NBUF)
        for j in range(nj - 3, nj):
            wait_out(j % _NBUF)

    return sc_gather, nw, nj


def _ln_body(x_ref, sid_ref, pos_ref, dseg_ref, w_ref, b_ref, o_ref, seg_ref):
    dseg = dseg_ref[...]
    for z in range(_ZB):
        sg = sid_ref[z]
        seg_ref[pl.ds(z * _K, _K), :] = lax.dot_general(
            sg, dseg, (((0,), (0,)), ((), ())),
            preferred_element_type=jnp.float32,
        )
    e = x_ref[...] + pos_ref[...] + seg_ref[...]
    mu = jnp.mean(e, axis=-1, keepdims=True)
    d = e - mu
    var = jnp.mean(d * d, axis=-1, keepdims=True)
    inv = lax.rsqrt(var + _EPS)
    o_ref[...] = d * inv * w_ref[0][None, :] + b_ref[0][None, :]


def kernel(token_ids, segment_ids, word_emb, pos_emb, seg_emb, ln_w, ln_b):
    bsz, seq = token_ids.shape
    vocab, hid = word_emb.shape
    nseg = seg_emb.shape[0]
    assert nseg == 2
    tok = bsz * seq

    info = plsc.get_sparse_core_info()
    nc, ns = info.num_cores, info.num_subcores
    sc_gather, nw, nj = _build_sc_gather(vocab, hid, tok, nc, ns)

    idx = token_ids.astype(jnp.int32).reshape(nw, nj, _K)
    gathered = sc_gather(word_emb, idx)

    g = tok // _K
    tb = _ZB * _K
    assert g % _ZB == 0 and tb % seq == 0
    posseg0 = pos_emb[:seq] + seg_emb[0][None, :]
    pos2 = jnp.tile(posseg0, (tb // seq, 1))
    sid3 = segment_ids.astype(jnp.float32).reshape(g, 1, _K)
    dseg = (seg_emb[1] - seg_emb[0]).reshape(1, hid)

    out = pl.pallas_call(
        _ln_body,
        grid=(tok // tb,),
        in_specs=[
            pl.BlockSpec((tb, hid), lambda i: (i, 0)),
            pl.BlockSpec((_ZB, 1, _K), lambda i: (i, 0, 0)),
            pl.BlockSpec((tb, hid), lambda i: (0, 0)),
            pl.BlockSpec((1, hid), lambda i: (0, 0)),
            pl.BlockSpec((1, hid), lambda i: (0, 0)),
            pl.BlockSpec((1, hid), lambda i: (0, 0)),
        ],
        out_specs=pl.BlockSpec((tb, hid), lambda i: (i, 0)),
        out_shape=jax.ShapeDtypeStruct((tok, hid), jnp.float32),
        scratch_shapes=[pltpu.VMEM((tb, hid), jnp.float32)],
        compiler_params=pltpu.CompilerParams(dimension_semantics=("parallel",)),
    )(
        gathered,
        sid3,
        pos2,
        dseg,
        ln_w.reshape(1, hid),
        ln_b.reshape(1, hid),
    )
    return out.reshape(bsz, seq, hid)


# speedup vs baseline: 1.7931x; 1.0764x over previous
import functools

import jax
import jax.numpy as jnp
from jax import lax
from jax.experimental import pallas as pl
from jax.experimental.pallas import tpu as pltpu
from jax.experimental.pallas import tpu_sc as plsc

_EPS = 1e-12
_K = 128
_NBUF = 4
_ZB = 50


def _build_sc_gather(vocab, hid, tok, nc, ns):
    nw = nc * ns
    per_w = tok // nw
    nj = per_w // _K
    assert per_w % _K == 0 and nj >= _NBUF
    nfull = (nj // _NBUF) * _NBUF

    mesh = plsc.VectorSubcoreMesh(core_axis_name="c", subcore_axis_name="s")

    @functools.partial(
        pl.kernel,
        mesh=mesh,
        out_type=jax.ShapeDtypeStruct((tok, hid), jnp.float32),
        scratch_types=[
            pltpu.VMEM((nj, _K), jnp.int32),
            [pltpu.VMEM((_K, hid), jnp.float32) for _ in range(_NBUF)],
            [pltpu.SemaphoreType.DMA for _ in range(_NBUF)],
            [pltpu.SemaphoreType.DMA for _ in range(_NBUF)],
        ],
    )
    def sc_gather(table, idx, out, idx_v, bufs, gsems, osems):
        wid = lax.axis_index("s") * nc + lax.axis_index("c")
        base = wid * per_w
        pltpu.sync_copy(idx.at[wid], idx_v)

        def start_gather(j, b):
            pltpu.make_async_copy(table.at[idx_v.at[j]], bufs[b], gsems[b]).start()

        def wait_gather(b):
            pltpu.make_async_copy(table.at[idx_v.at[0]], bufs[b], gsems[b]).wait()

        def start_out(j, b):
            pltpu.make_async_copy(
                bufs[b], out.at[pl.ds(base + j * _K, _K)], osems[b]
            ).start()

        def wait_out(b):
            pltpu.make_async_copy(
                bufs[b], out.at[pl.ds(base, _K)], osems[b]
            ).wait()

        def half_step(j, b):
            ab = (b + _NBUF - 1) % _NBUF
            gb = (b + _NBUF - 2) % _NBUF
            wait_gather(b)
            start_out(j, b)

            @pl.when(jnp.logical_and(j >= 1, j + 2 < nj))
            def _():
                wait_out(ab)

            @pl.when(j + 2 < nj)
            def _():
                start_gather(j + 2, gb)

        for b in range(2):
            start_gather(b, b)

        def step(i, carry):
            for b in range(_NBUF):
                half_step(i * _NBUF + b, b)
            return carry

        lax.fori_loop(0, nfull // _NBUF, step, 0)
        for j in range(nfull, nj):
            half_step(j, j % _NBUF)
        for j in range(nj - 3, nj):
            wait_out(j % _NBUF)

    return sc_gather, nw, nj


def _ln_body(x_ref, sid_ref, pos_ref, dseg_ref, w_ref, b_ref, o_ref, seg_ref):
    dseg = dseg_ref[...]
    for z in range(_ZB):
        sg = sid_ref[z]
        seg_ref[pl.ds(z * _K, _K), :] = lax.dot_general(
            sg, dseg, (((0,), (0,)), ((), ())),
            preferred_element_type=jnp.float32,
        )
    e = x_ref[...] + pos_ref[...] + seg_ref[...]
    mu = jnp.mean(e, axis=-1, keepdims=True)
    d = e - mu
    var = jnp.mean(d * d, axis=-1, keepdims=True)
    inv = lax.rsqrt(var + _EPS)
    o_ref[...] = d * inv * w_ref[0][None, :] + b_ref[0][None, :]


def kernel(token_ids, segment_ids, word_emb, pos_emb, seg_emb, ln_w, ln_b):
    bsz, seq = token_ids.shape
    vocab, hid = word_emb.shape
    nseg = seg_emb.shape[0]
    assert nseg == 2
    tok = bsz * seq

    info = plsc.get_sparse_core_info()
    nc, ns = info.num_cores, info.num_subcores
    sc_gather, nw, nj = _build_sc_gather(vocab, hid, tok, nc, ns)

    idx = token_ids.astype(jnp.int32).reshape(nw, nj, _K)
    gathered = sc_gather(word_emb, idx)

    g = tok // _K
    tb = _ZB * _K
    assert g % _ZB == 0 and tb % seq == 0
    posseg0 = pos_emb[:seq] + seg_emb[0][None, :]
    pos2 = jnp.tile(posseg0, (tb // seq, 1))
    sid3 = segment_ids.astype(jnp.float32).reshape(g, 1, _K)
    dseg = (seg_emb[1] - seg_emb[0]).reshape(1, hid)

    out = pl.pallas_call(
        _ln_body,
        grid=(tok // tb,),
        in_specs=[
            pl.BlockSpec((tb, hid), lambda i: (i, 0)),
            pl.BlockSpec((_ZB, 1, _K), lambda i: (i, 0, 0)),
            pl.BlockSpec((tb, hid), lambda i: (0, 0)),
            pl.BlockSpec((1, hid), lambda i: (0, 0)),
            pl.BlockSpec((1, hid), lambda i: (0, 0)),
            pl.BlockSpec((1, hid), lambda i: (0, 0)),
        ],
        out_specs=pl.BlockSpec((tb, hid), lambda i: (i, 0)),
        out_shape=jax.ShapeDtypeStruct((tok, hid), jnp.float32),
        scratch_shapes=[pltpu.VMEM((tb, hid), jnp.float32)],
        compiler_params=pltpu.CompilerParams(dimension_semantics=("parallel",)),
    )(
        gathered,
        sid3,
        pos2,
        dseg,
        ln_w.reshape(1, hid),
        ln_b.reshape(1, hid),
    )
    return out.reshape(bsz, seq, hid)


# speedup vs baseline: 1.8349x; 1.0233x over previous
import functools

import jax
import jax.numpy as jnp
from jax import lax
from jax.experimental import pallas as pl
from jax.experimental.pallas import tpu as pltpu
from jax.experimental.pallas import tpu_sc as plsc

_EPS = 1e-12
_K = 128
_NBUF = 4
_ZB = 100


def _build_sc_gather(vocab, hid, tok, nc, ns):
    nw = nc * ns
    per_w = tok // nw
    nj = per_w // _K
    assert per_w % _K == 0 and nj >= _NBUF
    nfull = (nj // _NBUF) * _NBUF

    mesh = plsc.VectorSubcoreMesh(core_axis_name="c", subcore_axis_name="s")

    @functools.partial(
        pl.kernel,
        mesh=mesh,
        out_type=jax.ShapeDtypeStruct((tok, hid), jnp.float32),
        scratch_types=[
            pltpu.VMEM((nj, _K), jnp.int32),
            [pltpu.VMEM((_K, hid), jnp.float32) for _ in range(_NBUF)],
            [pltpu.SemaphoreType.DMA for _ in range(_NBUF)],
            [pltpu.SemaphoreType.DMA for _ in range(_NBUF)],
        ],
    )
    def sc_gather(table, idx, out, idx_v, bufs, gsems, osems):
        wid = lax.axis_index("s") * nc + lax.axis_index("c")
        base = wid * per_w
        pltpu.sync_copy(idx.at[wid], idx_v)

        def start_gather(j, b):
            pltpu.make_async_copy(table.at[idx_v.at[j]], bufs[b], gsems[b]).start()

        def wait_gather(b):
            pltpu.make_async_copy(table.at[idx_v.at[0]], bufs[b], gsems[b]).wait()

        def start_out(j, b):
            pltpu.make_async_copy(
                bufs[b], out.at[pl.ds(base + j * _K, _K)], osems[b]
            ).start()

        def wait_out(b):
            pltpu.make_async_copy(
                bufs[b], out.at[pl.ds(base, _K)], osems[b]
            ).wait()

        def half_step(j, b):
            ab = (b + _NBUF - 1) % _NBUF
            gb = (b + _NBUF - 2) % _NBUF
            wait_gather(b)
            start_out(j, b)

            @pl.when(jnp.logical_and(j >= 1, j + 2 < nj))
            def _():
                wait_out(ab)

            @pl.when(j + 2 < nj)
            def _():
                start_gather(j + 2, gb)

        for b in range(2):
            start_gather(b, b)

        def step(i, carry):
            for b in range(_NBUF):
                half_step(i * _NBUF + b, b)
            return carry

        lax.fori_loop(0, nfull // _NBUF, step, 0)
        for j in range(nfull, nj):
            half_step(j, j % _NBUF)
        for j in range(nj - 3, nj):
            wait_out(j % _NBUF)

    return sc_gather, nw, nj


def _ln_body(x_ref, sid_ref, pos_ref, dseg_ref, w_ref, b_ref, o_ref, seg_ref):
    dseg = dseg_ref[...]
    for z in range(_ZB):
        sg = sid_ref[z]
        seg_ref[pl.ds(z * _K, _K), :] = lax.dot_general(
            sg, dseg, (((0,), (0,)), ((), ())),
            preferred_element_type=jnp.float32,
        )
    e = x_ref[...] + pos_ref[...] + seg_ref[...]
    mu = jnp.mean(e, axis=-1, keepdims=True)
    d = e - mu
    var = jnp.mean(d * d, axis=-1, keepdims=True)
    inv = lax.rsqrt(var + _EPS)
    o_ref[...] = d * inv * w_ref[0][None, :] + b_ref[0][None, :]


def kernel(token_ids, segment_ids, word_emb, pos_emb, seg_emb, ln_w, ln_b):
    bsz, seq = token_ids.shape
    vocab, hid = word_emb.shape
    nseg = seg_emb.shape[0]
    assert nseg == 2
    tok = bsz * seq

    info = plsc.get_sparse_core_info()
    nc, ns = info.num_cores, info.num_subcores
    sc_gather, nw, nj = _build_sc_gather(vocab, hid, tok, nc, ns)

    idx = token_ids.astype(jnp.int32).reshape(nw, nj, _K)
    gathered = sc_gather(word_emb, idx)

    g = tok // _K
    tb = _ZB * _K
    assert g % _ZB == 0 and tb % seq == 0
    posseg0 = pos_emb[:seq] + seg_emb[0][None, :]
    pos2 = jnp.tile(posseg0, (tb // seq, 1))
    sid3 = segment_ids.astype(jnp.float32).reshape(g, 1, _K)
    dseg = (seg_emb[1] - seg_emb[0]).reshape(1, hid)

    out = pl.pallas_call(
        _ln_body,
        grid=(tok // tb,),
        in_specs=[
            pl.BlockSpec((tb, hid), lambda i: (i, 0)),
            pl.BlockSpec((_ZB, 1, _K), lambda i: (i, 0, 0)),
            pl.BlockSpec((tb, hid), lambda i: (0, 0)),
            pl.BlockSpec((1, hid), lambda i: (0, 0)),
            pl.BlockSpec((1, hid), lambda i: (0, 0)),
            pl.BlockSpec((1, hid), lambda i: (0, 0)),
        ],
        out_specs=pl.BlockSpec((tb, hid), lambda i: (i, 0)),
        out_shape=jax.ShapeDtypeStruct((tok, hid), jnp.float32),
        scratch_shapes=[pltpu.VMEM((tb, hid), jnp.float32)],
        compiler_params=pltpu.CompilerParams(dimension_semantics=("parallel",)),
    )(
        gathered,
        sid3,
        pos2,
        dseg,
        ln_w.reshape(1, hid),
        ln_b.reshape(1, hid),
    )
    return out.reshape(bsz, seq, hid)
